# baseline (speedup 1.0000x reference)
import functools
import math

import jax
import jax.numpy as jnp
from jax import lax
from jax.experimental import pallas as pl
from jax.experimental.pallas import tpu as pltpu
from jax.experimental.pallas import tpu_sc as plsc


def kernel(indices, table, discrete_indices, offsets):
    B, S = indices.shape
    V, D = table.shape
    P = discrete_indices.shape[0]

    info = plsc.get_sparse_core_info()
    NC, NS, L = info.num_cores, info.num_subcores, info.num_lanes
    NW = NC * NS
    DG = D // 8
    BT = B // 128
    BW = B // NW
    TW = BW // 128
    assert B % (128 * NW) == 0 and D % 8 == 0

    mesh = plsc.VectorSubcoreMesh(core_axis_name="c", subcore_axis_name="s")

    @functools.partial(
        pl.kernel,
        mesh=mesh,
        out_type=jax.ShapeDtypeStruct((S, DG, BT, 8, 128), jnp.float32),
        compiler_params=pltpu.CompilerParams(
            use_tc_tiling_on_sc=False, needs_layout_passes=False
        ),
        scratch_types=[
            pltpu.VMEM((S, BW), jnp.int32),
            pltpu.VMEM((128,), jnp.int32),
            pltpu.VMEM((P,), jnp.int32),
            pltpu.VMEM((P, D), jnp.float32),
            pltpu.VMEM((2, TW, 8, 128), jnp.float32),
            pltpu.SemaphoreType.DMA,
            pltpu.SemaphoreType.DMA,
            pltpu.SemaphoreType.DMA,
        ],
    )
    def run(idxt_hbm, table_hbm, disc_hbm, offs_hbm, out_hbm,
            pk_v, offs_v, disc_v, sub_v, stage_v, gsem, sem0, sem1):
        wid = lax.axis_index("c") * NS + lax.axis_index("s")
        b0 = wid * BW

        idx_cp = pltpu.async_copy(
            idxt_hbm.at[:, pl.ds(b0, BW)], pk_v, gsem
        )
        pltpu.sync_copy(disc_hbm, disc_v)
        pltpu.sync_copy(offs_hbm, offs_v.at[pl.ds(0, S)])

        sub_cp = pltpu.async_copy(table_hbm.at[disc_v], sub_v, gsem)

        idx_cp.wait()

        def mk_pk(i, _):
            s = i // (BW // L)
            jb = (i % (BW // L)) * L
            off = plsc.load_gather(offs_v, [jnp.full((L,), s, jnp.int32)])
            pk_v[s, pl.ds(jb, L)] = pk_v[s, pl.ds(jb, L)] + off
            return 0

        lax.fori_loop(0, S * (BW // L), mk_pk, 0, unroll=4)

        sub_cp.wait()

        def pair(i, _):
            s = i // DG
            dg = i % DG
            par_sel = i % 2
            for par in range(2):

                @pl.when(par_sel == par)
                def _go():
                    sem = sem0 if par == 0 else sem1

                    @pl.when(i > 1)
                    def _wait():
                        pltpu.make_async_copy(
                            stage_v.at[par],
                            out_hbm.at[0, 0, pl.ds(0, TW)],
                            sem,
                        ).wait()

                    st = stage_v.at[par]
                    for bt in range(TW):
                        for b16 in range(8):
                            pk = pk_v[s, pl.ds(bt * 128 + b16 * 16, L)]
                            for dsub in range(8):
                                vals = plsc.load_gather(
                                    sub_v,
                                    [pk, jnp.full((L,), dg * 8 + dsub,
                                                  jnp.int32)],
                                )
                                st[bt, dsub, pl.ds(b16 * 16, L)] = vals
                    pltpu.make_async_copy(
                        st,
                        out_hbm.at[s, dg, pl.ds(b0 // 128, TW)],
                        sem,
                    ).start()

            return 0

        lax.fori_loop(0, S * DG, pair, 0)

        for sem in (sem0, sem1):
            pltpu.make_async_copy(
                stage_v.at[0], out_hbm.at[0, 0, pl.ds(0, TW)], sem
            ).wait()

    out5 = run(
        jnp.swapaxes(indices, 0, 1), table, discrete_indices, offsets
    )
    return out5.transpose(2, 4, 0, 1, 3).reshape(B, S, D)


# speedup vs baseline: 1.3879x; 1.3879x over previous
import functools
import math

import jax
import jax.numpy as jnp
from jax import lax
from jax.experimental import pallas as pl
from jax.experimental.pallas import tpu as pltpu
from jax.experimental.pallas import tpu_sc as plsc


def kernel(indices, table, discrete_indices, offsets):
    B, S = indices.shape
    V, D = table.shape
    P = discrete_indices.shape[0]

    info = plsc.get_sparse_core_info()
    NC, NS, L = info.num_cores, info.num_subcores, info.num_lanes
    NW = NC * NS
    DG = D // 8
    BT = B // 128
    BW = B // NW
    TW = BW // 128
    assert B % (128 * NW) == 0 and D % 8 == 0

    mesh = plsc.VectorSubcoreMesh(core_axis_name="c", subcore_axis_name="s")

    @functools.partial(
        pl.kernel,
        mesh=mesh,
        out_type=jax.ShapeDtypeStruct((S, DG, BT, 8, 128), jnp.float32),
        compiler_params=pltpu.CompilerParams(
            use_tc_tiling_on_sc=False, needs_layout_passes=False
        ),
        scratch_types=[
            pltpu.VMEM((S, BW), jnp.int32),
            pltpu.VMEM((128,), jnp.int32),
            pltpu.VMEM((P,), jnp.int32),
            pltpu.VMEM((P, D), jnp.float32),
            pltpu.VMEM((P, D + 1), jnp.float32),
            pltpu.VMEM((2, TW, 8, 128), jnp.float32),
            pltpu.SemaphoreType.DMA,
            pltpu.SemaphoreType.DMA,
            pltpu.SemaphoreType.DMA,
        ],
    )
    def run(idxt_hbm, table_hbm, disc_hbm, offs_hbm, out_hbm,
            pk_v, offs_v, disc_v, sub_v, sub65_v, stage_v, gsem, sem0, sem1):
        wid = lax.axis_index("c") * NS + lax.axis_index("s")
        b0 = wid * BW

        idx_cp = pltpu.async_copy(
            idxt_hbm.at[:, pl.ds(b0, BW)], pk_v, gsem
        )
        pltpu.sync_copy(disc_hbm, disc_v)
        pltpu.sync_copy(offs_hbm, offs_v.at[pl.ds(0, S)])

        sub_cp = pltpu.async_copy(table_hbm.at[disc_v], sub_v, gsem)

        idx_cp.wait()

        def mk_pk(i, _):
            s = i // (BW // L)
            jb = (i % (BW // L)) * L
            off = plsc.load_gather(offs_v, [jnp.full((L,), s, jnp.int32)])
            pk_v[s, pl.ds(jb, L)] = pk_v[s, pl.ds(jb, L)] + off
            return 0

        lax.fori_loop(0, S * (BW // L), mk_pk, 0, unroll=4)

        sub_cp.wait()

        def restride(p, _):
            for k in range(D // L):
                sub65_v[p, pl.ds(k * L, L)] = sub_v[p, pl.ds(k * L, L)]
            return 0

        lax.fori_loop(0, P, restride, 0, unroll=4)

        def pair(i, _):
            s = i // DG
            dg = i % DG
            par_sel = i % 2
            for par in range(2):

                @pl.when(par_sel == par)
                def _go():
                    sem = sem0 if par == 0 else sem1

                    @pl.when(i > 1)
                    def _wait():
                        pltpu.make_async_copy(
                            stage_v.at[par],
                            out_hbm.at[0, 0, pl.ds(0, TW)],
                            sem,
                        ).wait()

                    st = stage_v.at[par]
                    for bt in range(TW):
                        for b16 in range(8):
                            pk = pk_v[s, pl.ds(bt * 128 + b16 * 16, L)]
                            for dsub in range(8):
                                vals = plsc.load_gather(
                                    sub65_v,
                                    [pk, jnp.full((L,), dg * 8 + dsub,
                                                  jnp.int32)],
                                )
                                st[bt, dsub, pl.ds(b16 * 16, L)] = vals
                    pltpu.make_async_copy(
                        st,
                        out_hbm.at[s, dg, pl.ds(b0 // 128, TW)],
                        sem,
                    ).start()

            return 0

        lax.fori_loop(0, S * DG, pair, 0)

        for sem in (sem0, sem1):
            pltpu.make_async_copy(
                stage_v.at[0], out_hbm.at[0, 0, pl.ds(0, TW)], sem
            ).wait()

    out5 = run(
        jnp.swapaxes(indices, 0, 1), table, discrete_indices, offsets
    )
    return out5.transpose(2, 4, 0, 1, 3).reshape(B, S, D)


# speedup vs baseline: 1.6252x; 1.1710x over previous
import functools
import math

import jax
import jax.numpy as jnp
from jax import lax
from jax.experimental import pallas as pl
from jax.experimental.pallas import tpu as pltpu
from jax.experimental.pallas import tpu_sc as plsc


def _expand_rows(indices_flat, table, discrete_indices, offsets, N, D):
    S = offsets.shape[0]
    P = discrete_indices.shape[0]

    info = plsc.get_sparse_core_info()
    NC, NS, L = info.num_cores, info.num_subcores, info.num_lanes
    NW = NC * NS
    NR = N // NW
    C = 416
    NCHUNK = NR // C
    PERIOD = math.lcm(L, S)
    assert N % NW == 0 and NR % (2 * C) == 0 and C % L == 0
    assert NR % PERIOD == 0

    mesh = plsc.VectorSubcoreMesh(core_axis_name="c", subcore_axis_name="s")

    @functools.partial(
        pl.kernel,
        mesh=mesh,
        out_type=jax.ShapeDtypeStruct((N, D), jnp.float32),
        compiler_params=pltpu.CompilerParams(
            use_tc_tiling_on_sc=False, needs_layout_passes=False
        ),
        scratch_types=[
            pltpu.VMEM((NR,), jnp.int32),
            pltpu.VMEM((128,), jnp.int32),
            pltpu.VMEM((PERIOD,), jnp.int32),
            pltpu.VMEM((P,), jnp.int32),
            pltpu.VMEM_SHARED((P, D), jnp.float32),
            pltpu.VMEM((P, D), jnp.float32),
            pltpu.VMEM((2, C, D), jnp.float32),
            pltpu.SemaphoreType.DMA,
            pltpu.SemaphoreType.DMA,
            pltpu.SemaphoreType.DMA,
            pltpu.SemaphoreType.DMA,
        ],
    )
    def run(idx_hbm, table_hbm, disc_hbm, offs_hbm, out_hbm,
            idx_v, offs_v, pat_v, disc_v, sub_v, stage_v, rows_v,
            gsem, asem, sem0, sem1):
        wid = lax.axis_index("c") * NS + lax.axis_index("s")
        base = wid * NR

        pltpu.sync_copy(idx_hbm.at[pl.ds(base, NR)], idx_v)
        pltpu.sync_copy(disc_hbm, disc_v)
        pltpu.sync_copy(offs_hbm, offs_v.at[pl.ds(0, S)])

        @pl.when(lax.axis_index("s") == 0)
        def _build_sub():
            pltpu.async_copy(table_hbm.at[disc_v], stage_v, gsem).wait()
            pltpu.sync_copy(stage_v, sub_v)

        lane = lax.iota(jnp.int32, L)

        def mk_pat(i, _):
            jb = i * L
            sv = (jnp.full((L,), jb, jnp.int32) + lane) % S
            pat_v[pl.ds(jb, L)] = plsc.load_gather(offs_v, [sv])
            return 0

        lax.fori_loop(0, PERIOD // L, mk_pat, 0)

        NPAT = PERIOD // L

        def mk_packed(i, _):
            jb = i * L
            idx_v[pl.ds(jb, L)] = (
                idx_v[pl.ds(jb, L)] + pat_v[pl.ds((i % NPAT) * L, L)]
            )
            return 0

        lax.fori_loop(0, NR // L, mk_packed, 0, unroll=4)

        plsc.subcore_barrier()

        def two_chunks(h, _):
            for par in range(2):
                c = h * 2 + par
                sem = sem0 if par == 0 else sem1

                @pl.when(h > 0)
                def _wait():
                    pltpu.make_async_copy(
                        rows_v.at[par], out_hbm.at[pl.ds(0, C)], sem
                    ).wait()

                pltpu.async_copy(
                    sub_v.at[idx_v.at[pl.ds(c * C, C)]], rows_v.at[par], asem
                ).wait()
                pltpu.make_async_copy(
                    rows_v.at[par],
                    out_hbm.at[pl.ds(base + c * C, C)],
                    sem,
                ).start()
            return 0

        lax.fori_loop(0, NCHUNK // 2, two_chunks, 0)

        for sem in (sem0, sem1):
            pltpu.make_async_copy(
                rows_v.at[0], out_hbm.at[pl.ds(0, C)], sem
            ).wait()

    return run(indices_flat, table, discrete_indices, offsets)


def _to_batch_minor(lin, B, S, D):
    BLK_B = 512
    TBLK = BLK_B // 128
    DG = D // 8
    grid = (B // BLK_B,)

    def body(x_ref, o_ref):
        for s in range(S):
            xt = jnp.transpose(x_ref[:, s, :], (1, 0))
            y = xt.reshape(DG, 8, TBLK, 128)
            o_ref[s] = jnp.transpose(y, (0, 2, 1, 3))

    return pl.pallas_call(
        body,
        grid=grid,
        in_specs=[
            pl.BlockSpec((BLK_B, S, D), lambda bt: (bt, 0, 0)),
        ],
        out_specs=pl.BlockSpec(
            (S, DG, TBLK, 8, 128), lambda bt: (0, 0, bt, 0, 0)
        ),
        out_shape=jax.ShapeDtypeStruct((S, DG, B // 128, 8, 128),
                                       jnp.float32),
        compiler_params=pltpu.CompilerParams(
            dimension_semantics=("arbitrary",)
        ),
    )(lin)


def kernel(indices, table, discrete_indices, offsets):
    B, S = indices.shape
    V, D = table.shape
    N = B * S

    rows = _expand_rows(
        indices.reshape(N), table, discrete_indices, offsets, N, D
    )
    out5 = _to_batch_minor(rows.reshape(B, S, D), B, S, D)
    return out5.transpose(2, 4, 0, 1, 3).reshape(B, S, D)


# speedup vs baseline: 2.7592x; 1.6977x over previous
import functools
import math

import jax
import jax.numpy as jnp
from jax import lax
from jax.experimental import pallas as pl
from jax.experimental.pallas import tpu as pltpu
from jax.experimental.pallas import tpu_sc as plsc


def kernel(indices, table, discrete_indices, offsets):
    B, S = indices.shape
    V, D = table.shape
    P = discrete_indices.shape[0]

    info = plsc.get_sparse_core_info()
    NC, NS, L = info.num_cores, info.num_subcores, info.num_lanes
    NW = NC * NS
    DG = D // 8
    BT = B // 128
    BW = B // NW
    TW = BW // 128
    NT = S * DG * TW
    SD = D + 1
    assert B % (128 * NW) == 0 and D % 8 == 0

    mesh = plsc.VectorSubcoreMesh(core_axis_name="c", subcore_axis_name="s")

    @functools.partial(
        pl.kernel,
        mesh=mesh,
        out_type=jax.ShapeDtypeStruct((S, DG, BT, 8, 128), jnp.float32),
        compiler_params=pltpu.CompilerParams(
            use_tc_tiling_on_sc=False, needs_layout_passes=False
        ),
        scratch_types=[
            pltpu.VMEM((S, BW), jnp.int32),
            pltpu.VMEM((128,), jnp.int32),
            pltpu.VMEM((P,), jnp.int32),
            pltpu.VMEM((P, D), jnp.float32),
            pltpu.VMEM((P * SD,), jnp.float32),
            pltpu.VMEM((4, 8, 128), jnp.float32),
            pltpu.SemaphoreType.DMA,
            pltpu.SemaphoreType.DMA,
            pltpu.SemaphoreType.DMA,
        ],
    )
    def run(idxt_hbm, table_hbm, disc_hbm, offs_hbm, out_hbm,
            pk_v, offs_v, disc_v, sub_v, subp_v, ring_v, gsem, sem0, sem1):
        wid = lax.axis_index("c") * NS + lax.axis_index("s")
        b0 = wid * BW

        idx_cp = pltpu.async_copy(idxt_hbm.at[:, pl.ds(b0, BW)], pk_v, gsem)
        pltpu.sync_copy(disc_hbm, disc_v)
        pltpu.sync_copy(offs_hbm, offs_v.at[pl.ds(0, S)])

        sub_cp = pltpu.async_copy(table_hbm.at[disc_v], sub_v, gsem)

        idx_cp.wait()

        def mk_pk(i, _):
            s = i // (BW // L)
            jb = (i % (BW // L)) * L
            off = plsc.load_gather(offs_v, [jnp.full((L,), s, jnp.int32)])
            pk_v[s, pl.ds(jb, L)] = (pk_v[s, pl.ds(jb, L)] + off) * SD
            return 0

        lax.fori_loop(0, S * (BW // L), mk_pk, 0, unroll=4)

        sub_cp.wait()

        def restride(p, _):
            for k in range(D // L):
                subp_v[pl.ds(p * SD + k * L, L)] = sub_v[p, pl.ds(k * L, L)]
            return 0

        lax.fori_loop(0, P, restride, 0, unroll=4)

        def tile_loop(i, _):
            s = i // (DG * TW)
            rem = i % (DG * TW)
            dg = rem // TW
            t = rem % TW
            slot = i % 4

            @pl.when(jnp.logical_and(i >= 4, i % 2 == 0))
            def _wait0():
                pltpu.make_async_copy(
                    ring_v.at[0], out_hbm.at[0, 0, 0], sem0
                ).wait()

            @pl.when(jnp.logical_and(i >= 4, i % 2 == 1))
            def _wait1():
                pltpu.make_async_copy(
                    ring_v.at[0], out_hbm.at[0, 0, 0], sem1
                ).wait()

            for b16 in range(8):
                pk = pk_v[s, pl.ds(t * 128 + b16 * 16, L)]
                for dsub in range(8):
                    vals = plsc.load_gather(
                        subp_v, [pk + (dg * 8 + dsub)]
                    )
                    ring_v[slot, dsub, pl.ds(b16 * 16, L)] = vals

            @pl.when(i % 2 == 0)
            def _go0():
                pltpu.make_async_copy(
                    ring_v.at[slot],
                    out_hbm.at[s, dg, b0 // 128 + t],
                    sem0,
                ).start()

            @pl.when(i % 2 == 1)
            def _go1():
                pltpu.make_async_copy(
                    ring_v.at[slot],
                    out_hbm.at[s, dg, b0 // 128 + t],
                    sem1,
                ).start()

            return 0

        lax.fori_loop(0, NT, tile_loop, 0)

        for sem in (sem0, sem0, sem1, sem1):
            pltpu.make_async_copy(
                ring_v.at[0], out_hbm.at[0, 0, 0], sem
            ).wait()

    out5 = run(
        jnp.swapaxes(indices, 0, 1), table, discrete_indices, offsets
    )
    return out5.transpose(2, 4, 0, 1, 3).reshape(B, S, D)


# speedup vs baseline: 2.8310x; 1.0260x over previous
import functools
import math

import jax
import jax.numpy as jnp
from jax import lax
from jax.experimental import pallas as pl
from jax.experimental.pallas import tpu as pltpu
from jax.experimental.pallas import tpu_sc as plsc


def kernel(indices, table, discrete_indices, offsets):
    B, S = indices.shape
    V, D = table.shape
    P = discrete_indices.shape[0]

    info = plsc.get_sparse_core_info()
    NC, NS, L = info.num_cores, info.num_subcores, info.num_lanes
    NW = NC * NS
    DG = D // 8
    BT = B // 128
    BW = B // NW
    TW = BW // 128
    NT = S * DG * TW
    SD = D + 1
    assert B % (128 * NW) == 0 and D % 8 == 0

    mesh = plsc.VectorSubcoreMesh(core_axis_name="c", subcore_axis_name="s")

    @functools.partial(
        pl.kernel,
        mesh=mesh,
        out_type=jax.ShapeDtypeStruct((S, DG, BT, 8, 128), jnp.float32),
        compiler_params=pltpu.CompilerParams(
            use_tc_tiling_on_sc=False, needs_layout_passes=False
        ),
        scratch_types=[
            pltpu.VMEM((S, BW), jnp.int32),
            pltpu.VMEM((128,), jnp.int32),
            pltpu.VMEM((P,), jnp.int32),
            pltpu.VMEM((P, D), jnp.float32),
            pltpu.VMEM((P * SD,), jnp.float32),
            pltpu.VMEM((8, 8, 128), jnp.float32),
            pltpu.SemaphoreType.DMA,
            pltpu.SemaphoreType.DMA,
            pltpu.SemaphoreType.DMA,
        ],
    )
    def run(idxt_hbm, table_hbm, disc_hbm, offs_hbm, out_hbm,
            pk_v, offs_v, disc_v, sub_v, subp_v, ring_v, gsem, sem0, sem1):
        wid = lax.axis_index("c") * NS + lax.axis_index("s")
        b0 = wid * BW

        idx_cp = pltpu.async_copy(idxt_hbm.at[:, pl.ds(b0, BW)], pk_v, gsem)
        pltpu.sync_copy(disc_hbm, disc_v)
        pltpu.sync_copy(offs_hbm, offs_v.at[pl.ds(0, S)])

        sub_cp = pltpu.async_copy(table_hbm.at[disc_v], sub_v, gsem)

        idx_cp.wait()

        def mk_pk(i, _):
            s = i // (BW // L)
            jb = (i % (BW // L)) * L
            off = plsc.load_gather(offs_v, [jnp.full((L,), s, jnp.int32)])
            pk_v[s, pl.ds(jb, L)] = (pk_v[s, pl.ds(jb, L)] + off) * SD
            return 0

        lax.fori_loop(0, S * (BW // L), mk_pk, 0, unroll=4)

        sub_cp.wait()

        def restride(p, _):
            for k in range(D // L):
                subp_v[pl.ds(p * SD + k * L, L)] = sub_v[p, pl.ds(k * L, L)]
            return 0

        lax.fori_loop(0, P, restride, 0, unroll=4)

        def tile_loop(i, _):
            s = i // (DG * TW)
            rem = i % (DG * TW)
            dg = rem // TW
            t = rem % TW
            slot = i % 8

            @pl.when(jnp.logical_and(i >= 8, i % 2 == 0))
            def _wait0():
                pltpu.make_async_copy(
                    ring_v.at[0], out_hbm.at[0, 0, 0], sem0
                ).wait()

            @pl.when(jnp.logical_and(i >= 8, i % 2 == 1))
            def _wait1():
                pltpu.make_async_copy(
                    ring_v.at[0], out_hbm.at[0, 0, 0], sem1
                ).wait()

            for b16 in range(8):
                pk = pk_v[s, pl.ds(t * 128 + b16 * 16, L)]
                for dsub in range(8):
                    vals = plsc.load_gather(
                        subp_v, [pk + (dg * 8 + dsub)]
                    )
                    ring_v[slot, dsub, pl.ds(b16 * 16, L)] = vals

            @pl.when(i % 2 == 0)
            def _go0():
                pltpu.make_async_copy(
                    ring_v.at[slot],
                    out_hbm.at[s, dg, b0 // 128 + t],
                    sem0,
                ).start()

            @pl.when(i % 2 == 1)
            def _go1():
                pltpu.make_async_copy(
                    ring_v.at[slot],
                    out_hbm.at[s, dg, b0 // 128 + t],
                    sem1,
                ).start()

            return 0

        lax.fori_loop(0, NT, tile_loop, 0, unroll=2)

        for sem in (sem0, sem0, sem0, sem0, sem1, sem1, sem1, sem1):
            pltpu.make_async_copy(
                ring_v.at[0], out_hbm.at[0, 0, 0], sem
            ).wait()

    out5 = run(
        jnp.swapaxes(indices, 0, 1), table, discrete_indices, offsets
    )
    return out5.transpose(2, 4, 0, 1, 3).reshape(B, S, D)


# speedup vs baseline: 2.8322x; 1.0004x over previous
import functools

import jax
import jax.numpy as jnp
from jax import lax
from jax.experimental import pallas as pl
from jax.experimental.pallas import tpu as pltpu
from jax.experimental.pallas import tpu_sc as plsc


def kernel(indices, table, discrete_indices, offsets):
    B, S = indices.shape
    V, D = table.shape
    P = discrete_indices.shape[0]

    info = plsc.get_sparse_core_info()
    NC, NS, L = info.num_cores, info.num_subcores, info.num_lanes
    NW = NC * NS
    DG = D // 8
    BT = B // 128
    BW = B // NW
    TW = BW // 128
    NT = S * DG * TW
    SD = D + 1
    assert B % (128 * NW) == 0 and D % 8 == 0

    mesh = plsc.VectorSubcoreMesh(core_axis_name="c", subcore_axis_name="s")

    @functools.partial(
        pl.kernel,
        mesh=mesh,
        out_type=jax.ShapeDtypeStruct((S, DG, BT, 8, 128), jnp.float32),
        compiler_params=pltpu.CompilerParams(
            use_tc_tiling_on_sc=False, needs_layout_passes=False
        ),
        scratch_types=[
            pltpu.VMEM((S, BW), jnp.int32),
            pltpu.VMEM((128,), jnp.int32),
            pltpu.VMEM((P,), jnp.int32),
            pltpu.VMEM((P, D), jnp.float32),
            pltpu.VMEM((P * SD,), jnp.float32),
            pltpu.VMEM((8, 8, 128), jnp.float32),
            pltpu.SemaphoreType.DMA,
            pltpu.SemaphoreType.DMA,
            pltpu.SemaphoreType.DMA,
        ],
    )
    def run(idxt_hbm, table_hbm, disc_hbm, offs_hbm, out_hbm,
            pk_v, offs_v, disc_v, sub_v, subp_v, ring_v, gsem, sem0, sem1):
        wid = lax.axis_index("c") * NS + lax.axis_index("s")
        b0 = wid * BW

        idx_cp = pltpu.async_copy(idxt_hbm.at[:, pl.ds(b0, BW)], pk_v, gsem)
        pltpu.sync_copy(disc_hbm, disc_v)
        pltpu.sync_copy(offs_hbm, offs_v.at[pl.ds(0, S)])

        sub_cp = pltpu.async_copy(table_hbm.at[disc_v], sub_v, gsem)

        idx_cp.wait()

        def mk_pk(i, _):
            s = i // (BW // L)
            jb = (i % (BW // L)) * L
            off = plsc.load_gather(offs_v, [jnp.full((L,), s, jnp.int32)])
            pk_v[s, pl.ds(jb, L)] = (pk_v[s, pl.ds(jb, L)] + off) * SD
            return 0

        lax.fori_loop(0, S * (BW // L), mk_pk, 0, unroll=4)

        sub_cp.wait()

        def restride(p, _):
            for k in range(D // L):
                subp_v[pl.ds(p * SD + k * L, L)] = sub_v[p, pl.ds(k * L, L)]
            return 0

        lax.fori_loop(0, P, restride, 0, unroll=4)

        def tile_loop(i, _):
            s = i // (DG * TW)
            rem = i % (DG * TW)
            dg = rem // TW
            t = rem % TW
            slot = i % 8

            @pl.when(jnp.logical_and(i >= 8, i % 2 == 0))
            def _wait0():
                pltpu.make_async_copy(
                    ring_v.at[0], out_hbm.at[0, 0, 0], sem0
                ).wait()

            @pl.when(jnp.logical_and(i >= 8, i % 2 == 1))
            def _wait1():
                pltpu.make_async_copy(
                    ring_v.at[0], out_hbm.at[0, 0, 0], sem1
                ).wait()

            for b16 in range(8):
                pk = pk_v[s, pl.ds(t * 128 + b16 * 16, L)]
                for dsub in range(8):
                    vals = plsc.load_gather(
                        subp_v, [pk + (dg * 8 + dsub)]
                    )
                    ring_v[slot, dsub, pl.ds(b16 * 16, L)] = vals

            @pl.when(i % 2 == 0)
            def _go0():
                pltpu.make_async_copy(
                    ring_v.at[slot],
                    out_hbm.at[s, dg, b0 // 128 + t],
                    sem0,
                ).start()

            @pl.when(i % 2 == 1)
            def _go1():
                pltpu.make_async_copy(
                    ring_v.at[slot],
                    out_hbm.at[s, dg, b0 // 128 + t],
                    sem1,
                ).start()

            return 0

        lax.fori_loop(0, NT, tile_loop, 0, unroll=2)

        for sem in (sem0, sem0, sem0, sem0, sem1, sem1, sem1, sem1):
            pltpu.make_async_copy(
                ring_v.at[0], out_hbm.at[0, 0, 0], sem
            ).wait()

    out5 = run(
        jnp.swapaxes(indices, 0, 1), table, discrete_indices, offsets
    )
    return out5.transpose(2, 4, 0, 1, 3).reshape(B, S, D)


# speedup vs baseline: 5.6995x; 2.0124x over previous
import functools

import jax
import jax.numpy as jnp
from jax import lax
from jax.experimental import pallas as pl
from jax.experimental.pallas import tpu as pltpu
from jax.experimental.pallas import tpu_sc as plsc


def kernel(indices, table, discrete_indices, offsets):
    B, S = indices.shape
    V, D = table.shape
    P = discrete_indices.shape[0]

    info = plsc.get_sparse_core_info()
    NC, NS, L = info.num_cores, info.num_subcores, info.num_lanes
    NW = NC * NS
    DG = D // 8
    BT = B // 128
    BW = B // NW
    TW = BW // 128
    NT = S * DG * TW
    SD = D + 1
    assert B % (128 * NW) == 0 and D % 8 == 0

    mesh = plsc.VectorSubcoreMesh(core_axis_name="c", subcore_axis_name="s")

    @functools.partial(
        pl.kernel,
        mesh=mesh,
        out_type=jax.ShapeDtypeStruct((S, DG, BT, 8, 128), jnp.float32),
        compiler_params=pltpu.CompilerParams(
            use_tc_tiling_on_sc=False, needs_layout_passes=False
        ),
        scratch_types=[
            pltpu.VMEM((S, BW), jnp.int32),
            pltpu.VMEM((128,), jnp.int32),
            pltpu.VMEM((P,), jnp.int32),
            pltpu.VMEM((P, D), jnp.float32),
            pltpu.VMEM(((P + 16) * SD,), jnp.float32),
            pltpu.VMEM((8, 8, 128), jnp.float32),
            pltpu.SemaphoreType.DMA,
            pltpu.SemaphoreType.DMA,
            pltpu.SemaphoreType.DMA,
        ],
    )
    def run(idxt_hbm, table_hbm, disc_hbm, offs_hbm, out_hbm,
            pk_v, offs_v, disc_v, sub_v, subp_v, ring_v, gsem, sem0, sem1):
        wid = lax.axis_index("c") * NS + lax.axis_index("s")
        b0 = wid * BW
        lane = lax.iota(jnp.int32, L)

        idx_cp = pltpu.async_copy(idxt_hbm.at[:, pl.ds(b0, BW)], pk_v, gsem)
        pltpu.sync_copy(disc_hbm, disc_v)
        pltpu.sync_copy(offs_hbm, offs_v.at[pl.ds(0, S)])

        sub_cp = pltpu.async_copy(table_hbm.at[disc_v], sub_v, gsem)

        idx_cp.wait()
        sub_cp.wait()

        def restride(p, _):
            for k in range(D // L):
                subp_v[pl.ds(p * SD + k * L, L)] = sub_v[p, pl.ds(k * L, L)]
            return 0

        lax.fori_loop(0, P, restride, 0, unroll=4)

        def tile_loop(i, _):
            s = i // (DG * TW)
            rem = i % (DG * TW)
            dg = rem // TW
            t = rem % TW
            slot = i % 8

            @pl.when(jnp.logical_and(i >= 8, i % 2 == 0))
            def _wait0():
                pltpu.make_async_copy(
                    ring_v.at[0], out_hbm.at[0, 0, 0], sem0
                ).wait()

            @pl.when(jnp.logical_and(i >= 8, i % 2 == 1))
            def _wait1():
                pltpu.make_async_copy(
                    ring_v.at[0], out_hbm.at[0, 0, 0], sem1
                ).wait()

            off = plsc.load_gather(offs_v, [jnp.full((L,), s, jnp.int32)])
            wbase = (off + lane) * SD + dg * 8
            w = [plsc.load_gather(subp_v, [wbase + dsub])
                 for dsub in range(8)]
            for b16 in range(8):
                r = pk_v[s, pl.ds(t * 128 + b16 * 16, L)]
                for dsub in range(8):
                    vals = lax.gather(
                        w[dsub], r[:, None],
                        lax.GatherDimensionNumbers(
                            offset_dims=(),
                            collapsed_slice_dims=(0,),
                            start_index_map=(0,),
                        ),
                        slice_sizes=(1,),
                        mode=lax.GatherScatterMode.PROMISE_IN_BOUNDS,
                    )
                    ring_v[slot, dsub, pl.ds(b16 * 16, L)] = vals

            @pl.when(i % 2 == 0)
            def _go0():
                pltpu.make_async_copy(
                    ring_v.at[slot],
                    out_hbm.at[s, dg, b0 // 128 + t],
                    sem0,
                ).start()

            @pl.when(i % 2 == 1)
            def _go1():
                pltpu.make_async_copy(
                    ring_v.at[slot],
                    out_hbm.at[s, dg, b0 // 128 + t],
                    sem1,
                ).start()

            return 0

        lax.fori_loop(0, NT, tile_loop, 0, unroll=2)

        for sem in (sem0, sem0, sem0, sem0, sem1, sem1, sem1, sem1):
            pltpu.make_async_copy(
                ring_v.at[0], out_hbm.at[0, 0, 0], sem
            ).wait()

    out5 = run(
        jnp.swapaxes(indices, 0, 1), table, discrete_indices, offsets
    )
    return out5.transpose(2, 4, 0, 1, 3).reshape(B, S, D)


# speedup vs baseline: 5.7068x; 1.0013x over previous
import functools

import jax
import jax.numpy as jnp
from jax import lax
from jax.experimental import pallas as pl
from jax.experimental.pallas import tpu as pltpu
from jax.experimental.pallas import tpu_sc as plsc


def kernel(indices, table, discrete_indices, offsets):
    B, S = indices.shape
    V, D = table.shape
    P = discrete_indices.shape[0]

    info = plsc.get_sparse_core_info()
    NC, NS, L = info.num_cores, info.num_subcores, info.num_lanes
    NW = NC * NS
    DG = D // 8
    BT = B // 128
    BW = B // NW
    TW = BW // 128
    NT = S * DG * TW
    SD = D + 1
    assert B % (128 * NW) == 0 and D % 8 == 0

    mesh = plsc.VectorSubcoreMesh(core_axis_name="c", subcore_axis_name="s")

    @functools.partial(
        pl.kernel,
        mesh=mesh,
        out_type=jax.ShapeDtypeStruct((S, DG, BT, 8, 128), jnp.float32),
        compiler_params=pltpu.CompilerParams(
            use_tc_tiling_on_sc=False, needs_layout_passes=False
        ),
        scratch_types=[
            pltpu.VMEM((S, BW), jnp.int32),
            pltpu.VMEM((128,), jnp.int32),
            pltpu.VMEM((P,), jnp.int32),
            pltpu.VMEM((P, D), jnp.float32),
            pltpu.VMEM(((P + 16) * SD,), jnp.float32),
            pltpu.VMEM((8, 8, 128), jnp.float32),
            pltpu.SemaphoreType.DMA,
            pltpu.SemaphoreType.DMA,
            pltpu.SemaphoreType.DMA,
        ],
    )
    def run(idxt_hbm, table_hbm, disc_hbm, offs_hbm, out_hbm,
            pk_v, offs_v, disc_v, sub_v, subp_v, ring_v, gsem, sem0, sem1):
        wid = lax.axis_index("c") * NS + lax.axis_index("s")
        b0 = wid * BW
        lane = lax.iota(jnp.int32, L)

        idx_cp = pltpu.async_copy(idxt_hbm.at[:, pl.ds(b0, BW)], pk_v, gsem)
        pltpu.sync_copy(disc_hbm, disc_v)
        pltpu.sync_copy(offs_hbm, offs_v.at[pl.ds(0, S)])

        sub_cp = pltpu.async_copy(table_hbm.at[disc_v], sub_v, gsem)

        idx_cp.wait()
        sub_cp.wait()

        def restride(p, _):
            for k in range(D // L):
                subp_v[pl.ds(p * SD + k * L, L)] = sub_v[p, pl.ds(k * L, L)]
            return 0

        lax.fori_loop(0, P, restride, 0, unroll=4)

        def tile_loop(i, _):
            s = i // (DG * TW)
            rem = i % (DG * TW)
            dg = rem // TW
            t = rem % TW
            slot = i % 8

            @pl.when(jnp.logical_and(i >= 8, i % 2 == 0))
            def _wait0():
                pltpu.make_async_copy(
                    ring_v.at[0], out_hbm.at[0, 0, 0], sem0
                ).wait()

            @pl.when(jnp.logical_and(i >= 8, i % 2 == 1))
            def _wait1():
                pltpu.make_async_copy(
                    ring_v.at[0], out_hbm.at[0, 0, 0], sem1
                ).wait()

            off = plsc.load_gather(offs_v, [jnp.full((L,), s, jnp.int32)])
            wbase = (off + lane) * SD + dg * 8
            w = [plsc.load_gather(subp_v, [wbase + dsub])
                 for dsub in range(8)]
            for b16 in range(8):
                r = pk_v[s, pl.ds(t * 128 + b16 * 16, L)]
                for dsub in range(8):
                    vals = lax.gather(
                        w[dsub], r[:, None],
                        lax.GatherDimensionNumbers(
                            offset_dims=(),
                            collapsed_slice_dims=(0,),
                            start_index_map=(0,),
                        ),
                        slice_sizes=(1,),
                        mode=lax.GatherScatterMode.PROMISE_IN_BOUNDS,
                    )
                    ring_v[slot, dsub, pl.ds(b16 * 16, L)] = vals

            @pl.when(i % 2 == 0)
            def _go0():
                pltpu.make_async_copy(
                    ring_v.at[slot],
                    out_hbm.at[s, dg, b0 // 128 + t],
                    sem0,
                ).start()

            @pl.when(i % 2 == 1)
            def _go1():
                pltpu.make_async_copy(
                    ring_v.at[slot],
                    out_hbm.at[s, dg, b0 // 128 + t],
                    sem1,
                ).start()

            return 0

        lax.fori_loop(0, NT, tile_loop, 0, unroll=4)

        for sem in (sem0, sem0, sem0, sem0, sem1, sem1, sem1, sem1):
            pltpu.make_async_copy(
                ring_v.at[0], out_hbm.at[0, 0, 0], sem
            ).wait()

    out5 = run(
        jnp.swapaxes(indices, 0, 1), table, discrete_indices, offsets
    )
    return out5.transpose(2, 4, 0, 1, 3).reshape(B, S, D)
